# baseline (speedup 1.0000x reference)
import jax
import jax.numpy as jnp
from jax import lax
from jax.experimental import pallas as pl
from jax.experimental.pallas import tpu as pltpu
from jax.experimental.pallas import tpu_sc as plsc

NC, NS = 2, 16
NW = NC * NS
ROWS_PER_XFER = 128


def _tc_body(seqs_ref, cell_emb_ref, table_ref, w_ref, b_ref,
             cell_ref, comb_ref, idx_ref):
    bblk, L = seqs_ref.shape
    i = pl.program_id(0)

    cell = lax.dot_general(
        cell_emb_ref[...], w_ref[...],
        dimension_numbers=(((1,), (1,)), ((), ())),
        preferred_element_type=jnp.float32,
    ) + b_ref[...]
    cell_ref[...] = cell

    vocab = comb_ref.shape[1]
    comb_ref[...] = table_ref[:vocab][None, :, :] + cell[:, None, :]

    row = i * bblk + lax.broadcasted_iota(jnp.int32, (bblk, L), 0)
    idx_ref[...] = vocab * row + seqs_ref[...]


def _sc_body(comb_hbm, idx_hbm, out_hbm, idx_v, buf0, buf1, sem0, sem1):
    wid = lax.axis_index("s") * NC + lax.axis_index("c")
    n_xfer = idx_hbm.shape[1]
    base = wid * n_xfer * ROWS_PER_XFER

    pltpu.sync_copy(idx_hbm.at[wid], idx_v)

    pltpu.async_copy(comb_hbm.at[idx_v.at[0]], buf0, sem0)
    pltpu.async_copy(comb_hbm.at[idx_v.at[1]], buf1, sem1)

    def body(jj, carry):
        del carry
        pltpu.make_async_copy(comb_hbm.at[idx_v.at[0]], buf0, sem0).wait()
        j0 = 2 * jj
        pltpu.sync_copy(buf0, out_hbm.at[pl.ds(base + j0 * ROWS_PER_XFER,
                                               ROWS_PER_XFER)])

        @pl.when(j0 + 2 < n_xfer)
        def _():
            pltpu.async_copy(comb_hbm.at[idx_v.at[j0 + 2]], buf0, sem0)

        pltpu.make_async_copy(comb_hbm.at[idx_v.at[1]], buf1, sem1).wait()
        j1 = 2 * jj + 1
        pltpu.sync_copy(buf1, out_hbm.at[pl.ds(base + j1 * ROWS_PER_XFER,
                                               ROWS_PER_XFER)])

        @pl.when(j1 + 2 < n_xfer)
        def _():
            pltpu.async_copy(comb_hbm.at[idx_v.at[j1 + 2]], buf1, sem1)

        return 0

    lax.fori_loop(0, n_xfer // 2, body, 0)


def kernel(seqs, cell_emb, table, W, b):
    B, L = seqs.shape
    vocab, emb = table.shape
    cin = cell_emb.shape[1]

    vpad = 8
    table_p = jnp.zeros((vpad, emb), jnp.float32).at[:vocab].set(table)
    b2 = b.reshape(1, emb)

    BBLK = 512
    cell, comb, idx = pl.pallas_call(
        _tc_body,
        grid=(B // BBLK,),
        in_specs=[
            pl.BlockSpec((BBLK, L), lambda i: (i, 0)),
            pl.BlockSpec((BBLK, cin), lambda i: (i, 0)),
            pl.BlockSpec((vpad, emb), lambda i: (0, 0)),
            pl.BlockSpec((emb, cin), lambda i: (0, 0)),
            pl.BlockSpec((1, emb), lambda i: (0, 0)),
        ],
        out_specs=[
            pl.BlockSpec((BBLK, emb), lambda i: (i, 0)),
            pl.BlockSpec((BBLK, vocab, emb), lambda i: (i, 0, 0)),
            pl.BlockSpec((BBLK, L), lambda i: (i, 0)),
        ],
        out_shape=[
            jax.ShapeDtypeStruct((B, emb), jnp.float32),
            jax.ShapeDtypeStruct((B, vocab, emb), jnp.float32),
            jax.ShapeDtypeStruct((B, L), jnp.int32),
        ],
    )(seqs, cell_emb, table_p, W, b2)

    comb_flat = comb.reshape(B * vocab, emb)
    tokens = B * L
    n_xfer = tokens // (NW * ROWS_PER_XFER)
    idx3 = idx.reshape(NW, n_xfer, ROWS_PER_XFER)

    mesh = plsc.VectorSubcoreMesh(core_axis_name="c", subcore_axis_name="s")
    total_flat = pl.kernel(
        _sc_body,
        out_type=jax.ShapeDtypeStruct((tokens, emb), jnp.float32),
        mesh=mesh,
        scratch_types=[
            pltpu.VMEM((n_xfer, ROWS_PER_XFER), jnp.int32),
            pltpu.VMEM((ROWS_PER_XFER, emb), jnp.float32),
            pltpu.VMEM((ROWS_PER_XFER, emb), jnp.float32),
            pltpu.SemaphoreType.DMA,
            pltpu.SemaphoreType.DMA,
        ],
    )(comb_flat, idx3)

    return (total_flat.reshape(B, L, emb), cell)


# speedup vs baseline: 1.0941x; 1.0941x over previous
import jax
import jax.numpy as jnp
from jax import lax
from jax.experimental import pallas as pl
from jax.experimental.pallas import tpu as pltpu
from jax.experimental.pallas import tpu_sc as plsc

NC, NS = 2, 16
NW = NC * NS
ROWS_PER_XFER = 128


def _tc_body(seqs_ref, cell_emb_ref, table_ref, w_ref, b_ref,
             cell_ref, comb_ref, idx_ref):
    bblk, L = seqs_ref.shape
    i = pl.program_id(0)

    cell = lax.dot_general(
        cell_emb_ref[...], w_ref[...],
        dimension_numbers=(((1,), (1,)), ((), ())),
        preferred_element_type=jnp.float32,
    ) + b_ref[...]
    cell_ref[...] = cell

    vocab = comb_ref.shape[1]
    comb_ref[...] = table_ref[:vocab][None, :, :] + cell[:, None, :]

    row = i * bblk + lax.broadcasted_iota(jnp.int32, (bblk, L), 0)
    idx_ref[...] = vocab * row + seqs_ref[...]


NBUF = 4


def _sc_body(comb_hbm, idx_hbm, out_hbm, idx_v, *bufsems):
    bufs = bufsems[:NBUF]
    gsems = bufsems[NBUF:2 * NBUF]
    ssems = bufsems[2 * NBUF:]
    wid = lax.axis_index("s") * NC + lax.axis_index("c")
    n_xfer = idx_hbm.shape[1]
    base = wid * n_xfer * ROWS_PER_XFER

    pltpu.sync_copy(idx_hbm.at[wid], idx_v)

    for p in range(NBUF):
        pltpu.async_copy(comb_hbm.at[idx_v.at[p]], bufs[p], gsems[p])

    def body(jj, carry):
        del carry
        j0 = NBUF * jj
        for p in range(NBUF):
            pltpu.make_async_copy(comb_hbm.at[idx_v.at[p]], bufs[p],
                                  gsems[p]).wait()
            pltpu.async_copy(
                bufs[p],
                out_hbm.at[pl.ds(base + (j0 + p) * ROWS_PER_XFER,
                                 ROWS_PER_XFER)],
                ssems[p])
        for p in range(NBUF):
            jn = j0 + p + NBUF

            @pl.when(jn < n_xfer)
            def _(p=p, jn=jn):
                pltpu.make_async_copy(
                    bufs[p],
                    out_hbm.at[pl.ds(base, ROWS_PER_XFER)],
                    ssems[p]).wait()
                pltpu.async_copy(comb_hbm.at[idx_v.at[jn]], bufs[p], gsems[p])

        return 0

    lax.fori_loop(0, n_xfer // NBUF, body, 0)

    for p in range(NBUF):
        pltpu.make_async_copy(
            bufs[p], out_hbm.at[pl.ds(base, ROWS_PER_XFER)], ssems[p]).wait()


def kernel(seqs, cell_emb, table, W, b):
    B, L = seqs.shape
    vocab, emb = table.shape
    cin = cell_emb.shape[1]

    vpad = 8
    table_p = jnp.zeros((vpad, emb), jnp.float32).at[:vocab].set(table)
    b2 = b.reshape(1, emb)

    BBLK = 512
    cell, comb, idx = pl.pallas_call(
        _tc_body,
        grid=(B // BBLK,),
        in_specs=[
            pl.BlockSpec((BBLK, L), lambda i: (i, 0)),
            pl.BlockSpec((BBLK, cin), lambda i: (i, 0)),
            pl.BlockSpec((vpad, emb), lambda i: (0, 0)),
            pl.BlockSpec((emb, cin), lambda i: (0, 0)),
            pl.BlockSpec((1, emb), lambda i: (0, 0)),
        ],
        out_specs=[
            pl.BlockSpec((BBLK, emb), lambda i: (i, 0)),
            pl.BlockSpec((BBLK, vocab, emb), lambda i: (i, 0, 0)),
            pl.BlockSpec((BBLK, L), lambda i: (i, 0)),
        ],
        out_shape=[
            jax.ShapeDtypeStruct((B, emb), jnp.float32),
            jax.ShapeDtypeStruct((B, vocab, emb), jnp.float32),
            jax.ShapeDtypeStruct((B, L), jnp.int32),
        ],
    )(seqs, cell_emb, table_p, W, b2)

    comb_flat = comb.reshape(B * vocab, emb)
    tokens = B * L
    n_xfer = tokens // (NW * ROWS_PER_XFER)
    idx3 = idx.reshape(NW, n_xfer, ROWS_PER_XFER)

    mesh = plsc.VectorSubcoreMesh(core_axis_name="c", subcore_axis_name="s")
    total_flat = pl.kernel(
        _sc_body,
        out_type=jax.ShapeDtypeStruct((tokens, emb), jnp.float32),
        mesh=mesh,
        scratch_types=(
            [pltpu.VMEM((n_xfer, ROWS_PER_XFER), jnp.int32)]
            + [pltpu.VMEM((ROWS_PER_XFER, emb), jnp.float32)] * NBUF
            + [pltpu.SemaphoreType.DMA] * (2 * NBUF)
        ),
    )(comb_flat, idx3)

    return (total_flat.reshape(B, L, emb), cell)
